# Initial kernel scaffold; baseline (speedup 1.0000x reference)
import jax
import jax.numpy as jnp
from jax.experimental import pallas as pl


def kernel(hidden, W, b):
    raise NotImplementedError("write your pallas kernel here")


# speedup vs baseline: 2.3850x; 2.3850x over previous
import functools
import math

import jax
import jax.numpy as jnp
from jax.experimental import pallas as pl

_D = 2
_M_ANCH = 2
_NVE = 4
_K = 8
_E = 64
_GRID = (8, 8)
_EPS = 1e-6

_BN = 256


def _router_body(h_ref, wt_ref, b_ref, idx_ref, w_ref):
    h = h_ref[...]
    logits = jnp.dot(h, wt_ref[...], preferred_element_type=jnp.float32)
    logits = logits + b_ref[...]

    bn = logits.shape[0]

    def col(j):
        return logits[:, j:j + 1]

    anchor_cols = [(col(0), col(1), col(2)), (col(3), col(4), col(5))]

    a0 = anchor_cols[0][2]
    a1 = anchor_cols[1][2]
    mx = jnp.maximum(a0, a1)
    e0 = jnp.exp(a0 - mx)
    e1 = jnp.exp(a1 - mx)
    esum = e0 + e1
    pis = [e0 / esum, e1 / esum]

    lane_i = jax.lax.broadcasted_iota(jnp.int32, (bn, _E), 1)
    scale0 = float(_GRID[0] - 1)
    scale1 = float(_GRID[1] - 1)

    probs = jnp.zeros((bn, _E), dtype=jnp.float32)
    for m in range(_M_ANCH):
        c0, c1, _ = anchor_cols[m]
        u0 = jnp.clip(jax.nn.sigmoid(c0), _EPS, 1.0 - _EPS)
        u1 = jnp.clip(jax.nn.sigmoid(c1), _EPS, 1.0 - _EPS)
        p0 = jnp.maximum(jnp.minimum(u0 * scale0, scale0 - 1e-6), 0.0)
        p1 = jnp.maximum(jnp.minimum(u1 * scale1, scale1 - 1e-6), 0.0)
        af0 = jnp.clip(jnp.floor(p0), 0.0, scale0 - 1.0)
        af1 = jnp.clip(jnp.floor(p1), 0.0, scale1 - 1.0)
        f0 = jnp.clip(p0 - af0, _EPS, 1.0 - _EPS)
        f1 = jnp.clip(p1 - af1, _EPS, 1.0 - _EPS)
        ws = [(1.0 - f0) * (1.0 - f1), f0 * (1.0 - f1),
              (1.0 - f0) * f1, f0 * f1]
        wsum = ((ws[0] + ws[1]) + ws[2]) + ws[3]
        inv = pis[m] / (wsum + 1e-9)
        ai0 = af0.astype(jnp.int32)
        ai1 = af1.astype(jnp.int32)
        for t in range(_NVE):
            b0 = t & 1
            b1 = (t >> 1) & 1
            idx_i = (ai0 + b0) + 8 * (ai1 + b1)
            probs = probs + jnp.where(lane_i == idx_i, ws[t] * inv, 0.0)

    probs = jnp.maximum(probs, 0.0)
    psum = jnp.sum(probs, axis=1, keepdims=True)
    probs = probs / (psum + 1e-9)

    running = probs
    vals = []
    idxs = []
    for _ in range(_K):
        vmax = jnp.max(running, axis=1, keepdims=True)
        hit = running == vmax
        sel = jnp.min(jnp.where(hit, lane_i, _E), axis=1, keepdims=True)
        vals.append(vmax)
        idxs.append(sel)
        running = jnp.where(lane_i == sel, -1.0, running)

    idx_ref[...] = jnp.concatenate(idxs, axis=1)
    w_ref[...] = jnp.concatenate(vals, axis=1)


@jax.jit
def kernel(hidden, W, b):
    n, h = hidden.shape
    out_dim = W.shape[0]
    wt = jnp.zeros((h, 128), dtype=jnp.float32).at[:, :out_dim].set(W.T)
    bp = jnp.zeros((1, 128), dtype=jnp.float32).at[0, :out_dim].set(b)

    grid = (n // _BN,)
    top_idx, top_w = pl.pallas_call(
        _router_body,
        grid=grid,
        in_specs=[
            pl.BlockSpec((_BN, h), lambda i: (i, 0)),
            pl.BlockSpec((h, 128), lambda i: (0, 0)),
            pl.BlockSpec((1, 128), lambda i: (0, 0)),
        ],
        out_specs=[
            pl.BlockSpec((_BN, _K), lambda i: (i, 0)),
            pl.BlockSpec((_BN, _K), lambda i: (i, 0)),
        ],
        out_shape=[
            jax.ShapeDtypeStruct((n, _K), jnp.int32),
            jax.ShapeDtypeStruct((n, _K), jnp.float32),
        ],
    )(hidden, wt, bp)
    return (top_idx, top_w)


# speedup vs baseline: 3.6058x; 1.5119x over previous
import functools
import math

import jax
import jax.numpy as jnp
from jax.experimental import pallas as pl

_D = 2
_M_ANCH = 2
_NVE = 4
_K = 8
_E = 64
_GRID = (8, 8)
_EPS = 1e-6

_BN = 1024


def _router_body(h_ref, wt_ref, b_ref, idx_ref, w_ref):
    h = h_ref[...]
    logits = jnp.dot(h, wt_ref[...], preferred_element_type=jnp.float32)
    logits = logits + b_ref[...]

    bn = logits.shape[0]

    def col(j):
        return logits[:, j:j + 1]

    anchor_cols = [(col(0), col(1), col(2)), (col(3), col(4), col(5))]

    a0 = anchor_cols[0][2]
    a1 = anchor_cols[1][2]
    mx = jnp.maximum(a0, a1)
    e0 = jnp.exp(a0 - mx)
    e1 = jnp.exp(a1 - mx)
    esum = e0 + e1
    pis = [e0 / esum, e1 / esum]

    lane_i = jax.lax.broadcasted_iota(jnp.int32, (bn, _E), 1)
    scale0 = float(_GRID[0] - 1)
    scale1 = float(_GRID[1] - 1)

    probs = jnp.zeros((bn, _E), dtype=jnp.float32)
    for m in range(_M_ANCH):
        c0, c1, _ = anchor_cols[m]
        u0 = jnp.clip(jax.nn.sigmoid(c0), _EPS, 1.0 - _EPS)
        u1 = jnp.clip(jax.nn.sigmoid(c1), _EPS, 1.0 - _EPS)
        p0 = jnp.maximum(jnp.minimum(u0 * scale0, scale0 - 1e-6), 0.0)
        p1 = jnp.maximum(jnp.minimum(u1 * scale1, scale1 - 1e-6), 0.0)
        af0 = jnp.clip(jnp.floor(p0), 0.0, scale0 - 1.0)
        af1 = jnp.clip(jnp.floor(p1), 0.0, scale1 - 1.0)
        f0 = jnp.clip(p0 - af0, _EPS, 1.0 - _EPS)
        f1 = jnp.clip(p1 - af1, _EPS, 1.0 - _EPS)
        ws = [(1.0 - f0) * (1.0 - f1), f0 * (1.0 - f1),
              (1.0 - f0) * f1, f0 * f1]
        wsum = ((ws[0] + ws[1]) + ws[2]) + ws[3]
        inv = pis[m] / (wsum + 1e-9)
        ai0 = af0.astype(jnp.int32)
        ai1 = af1.astype(jnp.int32)
        for t in range(_NVE):
            b0 = t & 1
            b1 = (t >> 1) & 1
            idx_i = (ai0 + b0) + 8 * (ai1 + b1)
            probs = probs + jnp.where(lane_i == idx_i, ws[t] * inv, 0.0)

    probs = jnp.maximum(probs, 0.0)
    psum = jnp.sum(probs, axis=1, keepdims=True)
    probs = probs / (psum + 1e-9)

    running = probs
    vals = []
    idxs = []
    for _ in range(_K):
        vmax = jnp.max(running, axis=1, keepdims=True)
        hit = running == vmax
        sel = jnp.min(jnp.where(hit, lane_i, _E), axis=1, keepdims=True)
        vals.append(vmax)
        idxs.append(sel)
        running = jnp.where(lane_i == sel, -1.0, running)

    idx_ref[...] = jnp.concatenate(idxs, axis=1)
    w_ref[...] = jnp.concatenate(vals, axis=1)


@jax.jit
def kernel(hidden, W, b):
    n, h = hidden.shape
    out_dim = W.shape[0]
    wt = jnp.zeros((h, 128), dtype=jnp.float32).at[:, :out_dim].set(W.T)
    bp = jnp.zeros((1, 128), dtype=jnp.float32).at[0, :out_dim].set(b)

    grid = (n // _BN,)
    top_idx, top_w = pl.pallas_call(
        _router_body,
        grid=grid,
        in_specs=[
            pl.BlockSpec((_BN, h), lambda i: (i, 0)),
            pl.BlockSpec((h, 128), lambda i: (0, 0)),
            pl.BlockSpec((1, 128), lambda i: (0, 0)),
        ],
        out_specs=[
            pl.BlockSpec((_BN, _K), lambda i: (i, 0)),
            pl.BlockSpec((_BN, _K), lambda i: (i, 0)),
        ],
        out_shape=[
            jax.ShapeDtypeStruct((n, _K), jnp.int32),
            jax.ShapeDtypeStruct((n, _K), jnp.float32),
        ],
    )(hidden, wt, bp)
    return (top_idx, top_w)


# speedup vs baseline: 3.6061x; 1.0001x over previous
import functools
import math

import jax
import jax.numpy as jnp
from jax.experimental import pallas as pl

_D = 2
_M_ANCH = 2
_NVE = 4
_K = 8
_E = 64
_GRID = (8, 8)
_EPS = 1e-6

_BN = 2048


def _router_body(h_ref, wt_ref, b_ref, idx_ref, w_ref):
    h = h_ref[...]
    logits = jnp.dot(h, wt_ref[...], preferred_element_type=jnp.float32)
    logits = logits + b_ref[...]

    bn = logits.shape[0]

    def col(j):
        return logits[:, j:j + 1]

    anchor_cols = [(col(0), col(1), col(2)), (col(3), col(4), col(5))]

    a0 = anchor_cols[0][2]
    a1 = anchor_cols[1][2]
    mx = jnp.maximum(a0, a1)
    e0 = jnp.exp(a0 - mx)
    e1 = jnp.exp(a1 - mx)
    esum = e0 + e1
    pis = [e0 / esum, e1 / esum]

    lane_i = jax.lax.broadcasted_iota(jnp.int32, (bn, _E), 1)
    scale0 = float(_GRID[0] - 1)
    scale1 = float(_GRID[1] - 1)

    probs = jnp.zeros((bn, _E), dtype=jnp.float32)
    for m in range(_M_ANCH):
        c0, c1, _ = anchor_cols[m]
        u0 = jnp.clip(jax.nn.sigmoid(c0), _EPS, 1.0 - _EPS)
        u1 = jnp.clip(jax.nn.sigmoid(c1), _EPS, 1.0 - _EPS)
        p0 = jnp.maximum(jnp.minimum(u0 * scale0, scale0 - 1e-6), 0.0)
        p1 = jnp.maximum(jnp.minimum(u1 * scale1, scale1 - 1e-6), 0.0)
        af0 = jnp.clip(jnp.floor(p0), 0.0, scale0 - 1.0)
        af1 = jnp.clip(jnp.floor(p1), 0.0, scale1 - 1.0)
        f0 = jnp.clip(p0 - af0, _EPS, 1.0 - _EPS)
        f1 = jnp.clip(p1 - af1, _EPS, 1.0 - _EPS)
        ws = [(1.0 - f0) * (1.0 - f1), f0 * (1.0 - f1),
              (1.0 - f0) * f1, f0 * f1]
        wsum = ((ws[0] + ws[1]) + ws[2]) + ws[3]
        inv = pis[m] / (wsum + 1e-9)
        ai0 = af0.astype(jnp.int32)
        ai1 = af1.astype(jnp.int32)
        for t in range(_NVE):
            b0 = t & 1
            b1 = (t >> 1) & 1
            idx_i = (ai0 + b0) + 8 * (ai1 + b1)
            probs = probs + jnp.where(lane_i == idx_i, ws[t] * inv, 0.0)

    probs = jnp.maximum(probs, 0.0)
    psum = jnp.sum(probs, axis=1, keepdims=True)
    probs = probs / (psum + 1e-9)

    running = probs
    vals = []
    idxs = []
    for _ in range(_K):
        vmax = jnp.max(running, axis=1, keepdims=True)
        hit = running == vmax
        sel = jnp.min(jnp.where(hit, lane_i, _E), axis=1, keepdims=True)
        vals.append(vmax)
        idxs.append(sel)
        running = jnp.where(lane_i == sel, -1.0, running)

    idx_ref[...] = jnp.concatenate(idxs, axis=1)
    w_ref[...] = jnp.concatenate(vals, axis=1)


@jax.jit
def kernel(hidden, W, b):
    n, h = hidden.shape
    out_dim = W.shape[0]
    wt = jnp.zeros((h, 128), dtype=jnp.float32).at[:, :out_dim].set(W.T)
    bp = jnp.zeros((1, 128), dtype=jnp.float32).at[0, :out_dim].set(b)

    grid = (n // _BN,)
    top_idx, top_w = pl.pallas_call(
        _router_body,
        grid=grid,
        in_specs=[
            pl.BlockSpec((_BN, h), lambda i: (i, 0)),
            pl.BlockSpec((h, 128), lambda i: (0, 0)),
            pl.BlockSpec((1, 128), lambda i: (0, 0)),
        ],
        out_specs=[
            pl.BlockSpec((_BN, _K), lambda i: (i, 0)),
            pl.BlockSpec((_BN, _K), lambda i: (i, 0)),
        ],
        out_shape=[
            jax.ShapeDtypeStruct((n, _K), jnp.int32),
            jax.ShapeDtypeStruct((n, _K), jnp.float32),
        ],
    )(hidden, wt, bp)
    return (top_idx, top_w)


# speedup vs baseline: 11.7090x; 3.2470x over previous
import functools
import math

import jax
import jax.numpy as jnp
from jax.experimental import pallas as pl

_D = 2
_M_ANCH = 2
_NVE = 4
_K = 8
_E = 64
_GRID = (8, 8)
_EPS = 1e-6

_BN = 1024


def _router_body(w8_ref, b8_ref, h_ref, idx_ref, w_ref):
    lt = jax.lax.dot_general(
        w8_ref[...], h_ref[...],
        dimension_numbers=(((1,), (1,)), ((), ())),
        preferred_element_type=jnp.float32,
    )
    bn = lt.shape[1]
    lt = lt + b8_ref[...][:, 0:1]

    def row(j):
        return lt[j:j + 1, :]

    anchor_rows = [(row(0), row(1), row(2)), (row(3), row(4), row(5))]

    a0 = anchor_rows[0][2]
    a1 = anchor_rows[1][2]
    mx = jnp.maximum(a0, a1)
    e0 = jnp.exp(a0 - mx)
    e1 = jnp.exp(a1 - mx)
    esum = e0 + e1
    pis = [e0 / esum, e1 / esum]

    sub_i = jax.lax.broadcasted_iota(jnp.int32, (_E, bn), 0)
    scale0 = float(_GRID[0] - 1)
    scale1 = float(_GRID[1] - 1)

    probs = jnp.zeros((_E, bn), dtype=jnp.float32)
    for m in range(_M_ANCH):
        c0, c1, _ = anchor_rows[m]
        u0 = jnp.clip(jax.nn.sigmoid(c0), _EPS, 1.0 - _EPS)
        u1 = jnp.clip(jax.nn.sigmoid(c1), _EPS, 1.0 - _EPS)
        p0 = jnp.maximum(jnp.minimum(u0 * scale0, scale0 - 1e-6), 0.0)
        p1 = jnp.maximum(jnp.minimum(u1 * scale1, scale1 - 1e-6), 0.0)
        af0 = jnp.clip(jnp.floor(p0), 0.0, scale0 - 1.0)
        af1 = jnp.clip(jnp.floor(p1), 0.0, scale1 - 1.0)
        f0 = jnp.clip(p0 - af0, _EPS, 1.0 - _EPS)
        f1 = jnp.clip(p1 - af1, _EPS, 1.0 - _EPS)
        ws = [(1.0 - f0) * (1.0 - f1), f0 * (1.0 - f1),
              (1.0 - f0) * f1, f0 * f1]
        wsum = ((ws[0] + ws[1]) + ws[2]) + ws[3]
        inv = pis[m] / (wsum + 1e-9)
        ai0 = af0.astype(jnp.int32)
        ai1 = af1.astype(jnp.int32)
        for t in range(_NVE):
            b0 = t & 1
            b1 = (t >> 1) & 1
            idx_i = (ai0 + b0) + 8 * (ai1 + b1)
            probs = probs + jnp.where(sub_i == idx_i, ws[t] * inv, 0.0)

    probs = jnp.maximum(probs, 0.0)
    psum = jnp.sum(probs, axis=0, keepdims=True)
    probs = probs / (psum + 1e-9)

    running = probs
    vals = []
    idxs = []
    for _ in range(_K):
        vmax = jnp.max(running, axis=0, keepdims=True)
        hit = running == vmax
        sel = jnp.min(jnp.where(hit, sub_i, _E), axis=0, keepdims=True)
        vals.append(vmax)
        idxs.append(sel)
        running = jnp.where(sub_i == sel, -1.0, running)

    idx_ref[...] = jnp.concatenate(idxs, axis=0)
    w_ref[...] = jnp.concatenate(vals, axis=0)


@jax.jit
def kernel(hidden, W, b):
    n, h = hidden.shape
    out_dim = W.shape[0]
    w8 = jnp.zeros((8, h), dtype=jnp.float32).at[:out_dim, :].set(W)
    b8 = jnp.zeros((8, 128), dtype=jnp.float32).at[:out_dim, 0].set(b)

    grid = (n // _BN,)
    idx_t, w_t = pl.pallas_call(
        _router_body,
        grid=grid,
        in_specs=[
            pl.BlockSpec((8, h), lambda i: (0, 0)),
            pl.BlockSpec((8, 128), lambda i: (0, 0)),
            pl.BlockSpec((_BN, h), lambda i: (i, 0)),
        ],
        out_specs=[
            pl.BlockSpec((_K, _BN), lambda i: (0, i)),
            pl.BlockSpec((_K, _BN), lambda i: (0, i)),
        ],
        out_shape=[
            jax.ShapeDtypeStruct((_K, n), jnp.int32),
            jax.ShapeDtypeStruct((_K, n), jnp.float32),
        ],
    )(w8, b8, hidden)
    return (idx_t.T, w_t.T)


# speedup vs baseline: 12.1901x; 1.0411x over previous
import functools
import math

import jax
import jax.numpy as jnp
from jax.experimental import pallas as pl

_D = 2
_M_ANCH = 2
_NVE = 4
_K = 8
_E = 64
_GRID = (8, 8)
_EPS = 1e-6

_BN = 2048


def _router_body(w8_ref, b8_ref, h_ref, idx_ref, w_ref):
    lt = jax.lax.dot_general(
        w8_ref[...], h_ref[...],
        dimension_numbers=(((1,), (1,)), ((), ())),
        preferred_element_type=jnp.float32,
    )
    bn = lt.shape[1]
    lt = lt + b8_ref[...][:, 0:1]

    def row(j):
        return lt[j:j + 1, :]

    anchor_rows = [(row(0), row(1), row(2)), (row(3), row(4), row(5))]

    a0 = anchor_rows[0][2]
    a1 = anchor_rows[1][2]
    mx = jnp.maximum(a0, a1)
    e0 = jnp.exp(a0 - mx)
    e1 = jnp.exp(a1 - mx)
    esum = e0 + e1
    pis = [e0 / esum, e1 / esum]

    sub_i = jax.lax.broadcasted_iota(jnp.int32, (_E, bn), 0)
    scale0 = float(_GRID[0] - 1)
    scale1 = float(_GRID[1] - 1)

    probs = jnp.zeros((_E, bn), dtype=jnp.float32)
    for m in range(_M_ANCH):
        c0, c1, _ = anchor_rows[m]
        u0 = jnp.clip(jax.nn.sigmoid(c0), _EPS, 1.0 - _EPS)
        u1 = jnp.clip(jax.nn.sigmoid(c1), _EPS, 1.0 - _EPS)
        p0 = jnp.maximum(jnp.minimum(u0 * scale0, scale0 - 1e-6), 0.0)
        p1 = jnp.maximum(jnp.minimum(u1 * scale1, scale1 - 1e-6), 0.0)
        af0 = jnp.clip(jnp.floor(p0), 0.0, scale0 - 1.0)
        af1 = jnp.clip(jnp.floor(p1), 0.0, scale1 - 1.0)
        f0 = jnp.clip(p0 - af0, _EPS, 1.0 - _EPS)
        f1 = jnp.clip(p1 - af1, _EPS, 1.0 - _EPS)
        ws = [(1.0 - f0) * (1.0 - f1), f0 * (1.0 - f1),
              (1.0 - f0) * f1, f0 * f1]
        wsum = ((ws[0] + ws[1]) + ws[2]) + ws[3]
        inv = pis[m] / (wsum + 1e-9)
        ai0 = af0.astype(jnp.int32)
        ai1 = af1.astype(jnp.int32)
        for t in range(_NVE):
            b0 = t & 1
            b1 = (t >> 1) & 1
            idx_i = (ai0 + b0) + 8 * (ai1 + b1)
            probs = probs + jnp.where(sub_i == idx_i, ws[t] * inv, 0.0)

    probs = jnp.maximum(probs, 0.0)
    psum = jnp.sum(probs, axis=0, keepdims=True)
    probs = probs / (psum + 1e-9)

    running = probs
    vals = []
    idxs = []
    for _ in range(_K):
        vmax = jnp.max(running, axis=0, keepdims=True)
        hit = running == vmax
        sel = jnp.min(jnp.where(hit, sub_i, _E), axis=0, keepdims=True)
        vals.append(vmax)
        idxs.append(sel)
        running = jnp.where(sub_i == sel, -1.0, running)

    idx_ref[...] = jnp.concatenate(idxs, axis=0)
    w_ref[...] = jnp.concatenate(vals, axis=0)


@jax.jit
def kernel(hidden, W, b):
    n, h = hidden.shape
    out_dim = W.shape[0]
    w8 = jnp.zeros((8, h), dtype=jnp.float32).at[:out_dim, :].set(W)
    b8 = jnp.zeros((8, 128), dtype=jnp.float32).at[:out_dim, 0].set(b)

    grid = (n // _BN,)
    idx_t, w_t = pl.pallas_call(
        _router_body,
        grid=grid,
        in_specs=[
            pl.BlockSpec((8, h), lambda i: (0, 0)),
            pl.BlockSpec((8, 128), lambda i: (0, 0)),
            pl.BlockSpec((_BN, h), lambda i: (i, 0)),
        ],
        out_specs=[
            pl.BlockSpec((_K, _BN), lambda i: (0, i)),
            pl.BlockSpec((_K, _BN), lambda i: (0, i)),
        ],
        out_shape=[
            jax.ShapeDtypeStruct((_K, n), jnp.int32),
            jax.ShapeDtypeStruct((_K, n), jnp.float32),
        ],
    )(w8, b8, hidden)
    return (idx_t.T, w_t.T)
